# baseline (speedup 1.0000x reference)
import functools

import jax
import jax.numpy as jnp
from jax import lax
from jax.experimental import pallas as pl
from jax.experimental.pallas import tpu as pltpu, tpu_sc as plsc

N_NODES = 10000
N_EDGES = 320000
D_FEAT = 128
HIDDEN = 128
N_CLASSES = 40

NC = 2
NS = 16
LANES = 16

EDGES_PER_TILE = N_EDGES // (NC * NS)
KBLK = 16
NBLK = EDGES_PER_TILE // KBLK
NB_CH = 125
NCHUNK = NBLK // NB_CH
CH_EDGES = NB_CH * KBLK


@functools.cache
def _mesh():
    return plsc.VectorSubcoreMesh(
        core_axis_name="c", subcore_axis_name="s", num_cores=NC, num_subcores=NS
    )


def _deg_body(dst_hbm, out_hbm, dst_v, hist_v):
    c = lax.axis_index("c")
    s = lax.axis_index("s")
    w = c * NS + s
    base = pl.multiple_of(w * EDGES_PER_TILE, 8)
    pltpu.sync_copy(dst_hbm.at[pl.ds(base, EDGES_PER_TILE)], dst_v)
    zeros = jnp.zeros((LANES,), jnp.float32)

    def zbody(i, _):
        hist_v[pl.ds(i * LANES, LANES)] = zeros
        return 0

    lax.fori_loop(0, N_NODES // LANES, zbody, 0)
    ones = jnp.ones((LANES,), jnp.float32)

    def body(i, _):
        idx = dst_v[pl.ds(i * LANES, LANES)]
        plsc.addupdate_scatter(hist_v, [idx], ones)
        return 0

    lax.fori_loop(0, EDGES_PER_TILE // LANES, body, 0)
    pltpu.sync_copy(hist_v, out_hbm.at[w])


@functools.cache
def _deg_call():
    return pl.kernel(
        _deg_body,
        out_type=jax.ShapeDtypeStruct((NC * NS, N_NODES), jnp.float32),
        mesh=_mesh(),
        scratch_types=[
            pltpu.VMEM((EDGES_PER_TILE,), jnp.int32),
            pltpu.VMEM((N_NODES,), jnp.float32),
        ],
        compiler_params=pltpu.CompilerParams(needs_layout_passes=False),
    )


RCHUNK = 624
RLAST = N_NODES - (NS - 1) * RCHUNK


def _stage(s, src_view, dst_view):
    r0 = pl.multiple_of(s * RCHUNK, 8)

    @pl.when(s < NS - 1)
    def _():
        pltpu.sync_copy(src_view.at[pl.ds(r0, RCHUNK)],
                        dst_view.at[pl.ds(r0, RCHUNK)])

    @pl.when(s == NS - 1)
    def _():
        pltpu.sync_copy(src_view.at[pl.ds((NS - 1) * RCHUNK, RLAST)],
                        dst_view.at[pl.ds((NS - 1) * RCHUNK, RLAST)])


NBUF = 6
NTRI = NB_CH // NBUF - 1


def _agg_body(src_hbm, dst_hbm, y_hbm, out_hbm, src_v, dstb,
              gbuf0, gbuf1, gbuf2, gbuf3, gbuf4, gbuf5, acc_sh,
              gsem0, gsem1, gsem2, gsem3, gsem4, gsem5,
              ssem0, ssem1, ssem2, ssem3, ssem4, ssem5,
              dsem0, dsem1, dsem2, dsem3, dsem4, dsem5):
    c = lax.axis_index("c")
    s = lax.axis_index("s")
    gbufs = (gbuf0, gbuf1, gbuf2, gbuf3, gbuf4, gbuf5)
    gsems = (gsem0, gsem1, gsem2, gsem3, gsem4, gsem5)
    ssems = (ssem0, ssem1, ssem2, ssem3, ssem4, ssem5)
    dsems = (dsem0, dsem1, dsem2, dsem3, dsem4, dsem5)
    _stage(s, y_hbm, acc_sh)
    plsc.subcore_barrier()

    tile_base = (c * NS + s) * EDGES_PER_TILE

    def chunk(ch, _):
        chbase = pl.multiple_of(tile_base + ch * CH_EDGES, 8)
        pltpu.sync_copy(src_hbm.at[pl.ds(chbase, CH_EDGES)], src_v)

        def d_start(j, b):
            off = pl.multiple_of(chbase + j * KBLK, 8)
            pltpu.async_copy(dst_hbm.at[pl.ds(off, KBLK)],
                             dstb.at[b], dsems[b])

        def d_wait(b):
            pltpu.make_async_copy(dst_hbm.at[pl.ds(0, KBLK)],
                                  dstb.at[b], dsems[b]).wait()

        def g_start(j, b):
            idx = src_v.at[pl.ds(j * KBLK, KBLK)]
            pltpu.async_copy(y_hbm.at[idx], gbufs[b], gsems[b])

        def g_wait(b):
            idx = src_v.at[pl.ds(0, KBLK)]
            pltpu.make_async_copy(y_hbm.at[idx], gbufs[b], gsems[b]).wait()

        def s_start(b):
            pltpu.async_copy(gbufs[b], acc_sh.at[dstb.at[b]], ssems[b],
                             add=True)

        def s_wait(b):
            pltpu.make_async_copy(gbufs[b], acc_sh.at[dstb.at[0]],
                                  ssems[b]).wait()

        for b in range(NBUF):
            d_start(b, b)
            g_start(b, b)

        def triple(t, _):
            j = NBUF * t
            for b in range(NBUF):
                d_wait(b)
                g_wait(b)
                s_start(b)
                s_wait(b)
                d_start(j + NBUF + b, b)
                g_start(j + NBUF + b, b)
            return 0

        lax.fori_loop(0, NTRI, triple, 0)
        for b in range(NBUF):
            d_wait(b)
            g_wait(b)
            s_start(b)
            if b < NB_CH - NBUF * (NTRI + 1):
                s_wait(b)
                d_start(NBUF * (NTRI + 1) + b, b)
                g_start(NBUF * (NTRI + 1) + b, b)
        for b in range(NB_CH - NBUF * (NTRI + 1)):
            d_wait(b)
            g_wait(b)
            s_start(b)
        for b in range(NBUF):
            s_wait(b)
        return 0

    lax.fori_loop(0, NCHUNK, chunk, 0)
    plsc.subcore_barrier()
    _stage(s, acc_sh, out_hbm.at[c])


@functools.cache
def _agg_call():
    return pl.kernel(
        _agg_body,
        out_type=jax.ShapeDtypeStruct((NC, N_NODES, D_FEAT), jnp.float32),
        mesh=_mesh(),
        scratch_types=(
            [pltpu.VMEM((CH_EDGES,), jnp.int32)]
            + [pltpu.VMEM((NBUF, KBLK), jnp.int32)]
            + [pltpu.VMEM((KBLK, D_FEAT), jnp.float32)] * NBUF
            + [pltpu.MemorySpace.VMEM_SHARED((N_NODES, D_FEAT), jnp.float32)]
            + [pltpu.SemaphoreType.DMA] * (3 * NBUF)
        ),
    )


MBLK = 1000
GRID = N_NODES // MBLK


def _mm1_body(deg_ref, x_ref, w_ref, y_ref, dinv_ref):
    deg = jnp.sum(deg_ref[...], axis=1) + 1.0
    dinv = lax.rsqrt(deg)[:, None]
    xw = jnp.dot(x_ref[...], w_ref[...], preferred_element_type=jnp.float32)
    y_ref[...] = xw * dinv
    dinv_ref[...] = dinv


def _mid_body(agg_ref, y_ref, dinv_ref, b1_ref, w_ref, out_ref):
    a = agg_ref[0] + agg_ref[1] - y_ref[...]
    dinv = dinv_ref[...]
    h = jnp.maximum(a * dinv + b1_ref[...], 0.0)
    out_ref[...] = jnp.dot(h, w_ref[...], preferred_element_type=jnp.float32) * dinv


def _head_body(agg_ref, y_ref, dinv_ref, b2_ref, w3_ref, b3_ref, out_ref):
    a = agg_ref[0] + agg_ref[1] - y_ref[...]
    h = jnp.maximum(a * dinv_ref[...] + b2_ref[...], 0.0)
    out_ref[...] = (
        jnp.dot(h, w3_ref[...], preferred_element_type=jnp.float32) + b3_ref[...]
    )


def _mm1(degs, x, W1):
    return pl.pallas_call(
        _mm1_body,
        grid=(GRID,),
        in_specs=[
            pl.BlockSpec((MBLK, NC * NS), lambda i: (i, 0)),
            pl.BlockSpec((MBLK, D_FEAT), lambda i: (i, 0)),
            pl.BlockSpec((D_FEAT, HIDDEN), lambda i: (0, 0)),
        ],
        out_specs=[
            pl.BlockSpec((MBLK, HIDDEN), lambda i: (i, 0)),
            pl.BlockSpec((MBLK, 1), lambda i: (i, 0)),
        ],
        out_shape=[
            jax.ShapeDtypeStruct((N_NODES, HIDDEN), jnp.float32),
            jax.ShapeDtypeStruct((N_NODES, 1), jnp.float32),
        ],
    )(degs, x, W1)


def _mid(agg1, y1, dinv, b1, W2):
    return pl.pallas_call(
        _mid_body,
        grid=(GRID,),
        in_specs=[
            pl.BlockSpec((NC, MBLK, HIDDEN), lambda i: (0, i, 0)),
            pl.BlockSpec((MBLK, HIDDEN), lambda i: (i, 0)),
            pl.BlockSpec((MBLK, 1), lambda i: (i, 0)),
            pl.BlockSpec((1, HIDDEN), lambda i: (0, 0)),
            pl.BlockSpec((HIDDEN, HIDDEN), lambda i: (0, 0)),
        ],
        out_specs=pl.BlockSpec((MBLK, HIDDEN), lambda i: (i, 0)),
        out_shape=jax.ShapeDtypeStruct((N_NODES, HIDDEN), jnp.float32),
    )(agg1, y1, dinv, b1, W2)


def _head(agg2, y2, dinv, b2, W3, b3):
    return pl.pallas_call(
        _head_body,
        grid=(GRID,),
        in_specs=[
            pl.BlockSpec((NC, MBLK, HIDDEN), lambda i: (0, i, 0)),
            pl.BlockSpec((MBLK, HIDDEN), lambda i: (i, 0)),
            pl.BlockSpec((MBLK, 1), lambda i: (i, 0)),
            pl.BlockSpec((1, HIDDEN), lambda i: (0, 0)),
            pl.BlockSpec((HIDDEN, N_CLASSES), lambda i: (0, 0)),
            pl.BlockSpec((1, N_CLASSES), lambda i: (0, 0)),
        ],
        out_specs=pl.BlockSpec((MBLK, N_CLASSES), lambda i: (i, 0)),
        out_shape=jax.ShapeDtypeStruct((N_NODES, N_CLASSES), jnp.float32),
    )(agg2, y2, dinv, b2, W3, b3)


def kernel(x, edge_index, W1, b1, W2, b2, W3, b3):
    ei = edge_index.astype(jnp.int32)

    src1d = ei[0]
    dst1d = ei[1]
    degs = _deg_call()(dst1d)
    y1, dinv = _mm1(degs.T, x, W1)
    agg1 = _agg_call()(src1d, dst1d, y1)
    y2 = _mid(agg1, y1, dinv, b1.reshape(1, HIDDEN), W2)
    agg2 = _agg_call()(src1d, dst1d, y2)
    logits = _head(agg2, y2, dinv, b2.reshape(1, HIDDEN), W3,
                   b3.reshape(1, N_CLASSES))
    return logits


# speedup vs baseline: 1.0084x; 1.0084x over previous
import functools

import jax
import jax.numpy as jnp
from jax import lax
from jax.experimental import pallas as pl
from jax.experimental.pallas import tpu as pltpu, tpu_sc as plsc

N_NODES = 10000
N_EDGES = 320000
D_FEAT = 128
HIDDEN = 128
N_CLASSES = 40

NC = 2
NS = 16
LANES = 16

EDGES_PER_TILE = N_EDGES // (NC * NS)
KBLK = 40
NBLK = EDGES_PER_TILE // KBLK
NB_CH = 50
NCHUNK = NBLK // NB_CH
CH_EDGES = NB_CH * KBLK


@functools.cache
def _mesh():
    return plsc.VectorSubcoreMesh(
        core_axis_name="c", subcore_axis_name="s", num_cores=NC, num_subcores=NS
    )


def _deg_body(dst_hbm, out_hbm, dst_v, hist_v):
    c = lax.axis_index("c")
    s = lax.axis_index("s")
    w = c * NS + s
    base = pl.multiple_of(w * EDGES_PER_TILE, 8)
    pltpu.sync_copy(dst_hbm.at[pl.ds(base, EDGES_PER_TILE)], dst_v)
    zeros = jnp.zeros((LANES,), jnp.float32)

    def zbody(i, _):
        hist_v[pl.ds(i * LANES, LANES)] = zeros
        return 0

    lax.fori_loop(0, N_NODES // LANES, zbody, 0)
    ones = jnp.ones((LANES,), jnp.float32)

    def body(i, _):
        idx = dst_v[pl.ds(i * LANES, LANES)]
        plsc.addupdate_scatter(hist_v, [idx], ones)
        return 0

    lax.fori_loop(0, EDGES_PER_TILE // LANES, body, 0)
    pltpu.sync_copy(hist_v, out_hbm.at[w])


@functools.cache
def _deg_call():
    return pl.kernel(
        _deg_body,
        out_type=jax.ShapeDtypeStruct((NC * NS, N_NODES), jnp.float32),
        mesh=_mesh(),
        scratch_types=[
            pltpu.VMEM((EDGES_PER_TILE,), jnp.int32),
            pltpu.VMEM((N_NODES,), jnp.float32),
        ],
        compiler_params=pltpu.CompilerParams(needs_layout_passes=False),
    )


RCHUNK = 624
RLAST = N_NODES - (NS - 1) * RCHUNK


def _stage(s, src_view, dst_view):
    r0 = pl.multiple_of(s * RCHUNK, 8)

    @pl.when(s < NS - 1)
    def _():
        pltpu.sync_copy(src_view.at[pl.ds(r0, RCHUNK)],
                        dst_view.at[pl.ds(r0, RCHUNK)])

    @pl.when(s == NS - 1)
    def _():
        pltpu.sync_copy(src_view.at[pl.ds((NS - 1) * RCHUNK, RLAST)],
                        dst_view.at[pl.ds((NS - 1) * RCHUNK, RLAST)])


NBUF = 3
NTRI = NB_CH // NBUF - 1


def _agg_body(src_hbm, dst_hbm, y_hbm, out_hbm, src_v, dstb,
              gbuf0, gbuf1, gbuf2, acc_sh,
              gsem0, gsem1, gsem2, ssem0, ssem1, ssem2, dsem0, dsem1, dsem2):
    c = lax.axis_index("c")
    s = lax.axis_index("s")
    gbufs = (gbuf0, gbuf1, gbuf2)
    gsems = (gsem0, gsem1, gsem2)
    ssems = (ssem0, ssem1, ssem2)
    dsems = (dsem0, dsem1, dsem2)
    _stage(s, y_hbm, acc_sh)
    plsc.subcore_barrier()

    tile_base = (c * NS + s) * EDGES_PER_TILE

    def chunk(ch, _):
        chbase = pl.multiple_of(tile_base + ch * CH_EDGES, 8)
        pltpu.sync_copy(src_hbm.at[pl.ds(chbase, CH_EDGES)], src_v)

        def d_start(j, b):
            off = pl.multiple_of(chbase + j * KBLK, 8)
            pltpu.async_copy(dst_hbm.at[pl.ds(off, KBLK)],
                             dstb.at[b], dsems[b])

        def d_wait(b):
            pltpu.make_async_copy(dst_hbm.at[pl.ds(0, KBLK)],
                                  dstb.at[b], dsems[b]).wait()

        def g_start(j, b):
            idx = src_v.at[pl.ds(j * KBLK, KBLK)]
            pltpu.async_copy(y_hbm.at[idx], gbufs[b], gsems[b])

        def g_wait(b):
            idx = src_v.at[pl.ds(0, KBLK)]
            pltpu.make_async_copy(y_hbm.at[idx], gbufs[b], gsems[b]).wait()

        def s_start(b):
            pltpu.async_copy(gbufs[b], acc_sh.at[dstb.at[b]], ssems[b],
                             add=True)

        def s_wait(b):
            pltpu.make_async_copy(gbufs[b], acc_sh.at[dstb.at[0]],
                                  ssems[b]).wait()

        for b in range(NBUF):
            d_start(b, b)
            g_start(b, b)

        def triple(t, _):
            j = NBUF * t
            for b in range(NBUF):
                d_wait(b)
                g_wait(b)
                s_start(b)
                s_wait(b)
                d_start(j + NBUF + b, b)
                g_start(j + NBUF + b, b)
            return 0

        lax.fori_loop(0, NTRI, triple, 0)
        for b in range(NBUF):
            d_wait(b)
            g_wait(b)
            s_start(b)
            if b < NB_CH - NBUF * (NTRI + 1):
                s_wait(b)
                d_start(NBUF * (NTRI + 1) + b, b)
                g_start(NBUF * (NTRI + 1) + b, b)
        for b in range(NB_CH - NBUF * (NTRI + 1)):
            d_wait(b)
            g_wait(b)
            s_start(b)
        for b in range(NBUF):
            s_wait(b)
        return 0

    lax.fori_loop(0, NCHUNK, chunk, 0)
    plsc.subcore_barrier()
    _stage(s, acc_sh, out_hbm.at[c])


@functools.cache
def _agg_call():
    return pl.kernel(
        _agg_body,
        out_type=jax.ShapeDtypeStruct((NC, N_NODES, D_FEAT), jnp.float32),
        mesh=_mesh(),
        scratch_types=(
            [pltpu.VMEM((CH_EDGES,), jnp.int32)]
            + [pltpu.VMEM((NBUF, KBLK), jnp.int32)]
            + [pltpu.VMEM((KBLK, D_FEAT), jnp.float32)] * NBUF
            + [pltpu.MemorySpace.VMEM_SHARED((N_NODES, D_FEAT), jnp.float32)]
            + [pltpu.SemaphoreType.DMA] * (3 * NBUF)
        ),
    )


MBLK = 1000
GRID = N_NODES // MBLK


def _mm1_body(deg_ref, x_ref, w_ref, y_ref, dinv_ref):
    deg = jnp.sum(deg_ref[...], axis=1) + 1.0
    dinv = lax.rsqrt(deg)[:, None]
    xw = jnp.dot(x_ref[...], w_ref[...], preferred_element_type=jnp.float32)
    y_ref[...] = xw * dinv
    dinv_ref[...] = dinv


def _mid_body(agg_ref, y_ref, dinv_ref, b1_ref, w_ref, out_ref):
    a = agg_ref[0] + agg_ref[1] - y_ref[...]
    dinv = dinv_ref[...]
    h = jnp.maximum(a * dinv + b1_ref[...], 0.0)
    out_ref[...] = jnp.dot(h, w_ref[...], preferred_element_type=jnp.float32) * dinv


def _head_body(agg_ref, y_ref, dinv_ref, b2_ref, w3_ref, b3_ref, out_ref):
    a = agg_ref[0] + agg_ref[1] - y_ref[...]
    h = jnp.maximum(a * dinv_ref[...] + b2_ref[...], 0.0)
    out_ref[...] = (
        jnp.dot(h, w3_ref[...], preferred_element_type=jnp.float32) + b3_ref[...]
    )


def _mm1(degs, x, W1):
    return pl.pallas_call(
        _mm1_body,
        grid=(GRID,),
        in_specs=[
            pl.BlockSpec((MBLK, NC * NS), lambda i: (i, 0)),
            pl.BlockSpec((MBLK, D_FEAT), lambda i: (i, 0)),
            pl.BlockSpec((D_FEAT, HIDDEN), lambda i: (0, 0)),
        ],
        out_specs=[
            pl.BlockSpec((MBLK, HIDDEN), lambda i: (i, 0)),
            pl.BlockSpec((MBLK, 1), lambda i: (i, 0)),
        ],
        out_shape=[
            jax.ShapeDtypeStruct((N_NODES, HIDDEN), jnp.float32),
            jax.ShapeDtypeStruct((N_NODES, 1), jnp.float32),
        ],
    )(degs, x, W1)


def _mid(agg1, y1, dinv, b1, W2):
    return pl.pallas_call(
        _mid_body,
        grid=(GRID,),
        in_specs=[
            pl.BlockSpec((NC, MBLK, HIDDEN), lambda i: (0, i, 0)),
            pl.BlockSpec((MBLK, HIDDEN), lambda i: (i, 0)),
            pl.BlockSpec((MBLK, 1), lambda i: (i, 0)),
            pl.BlockSpec((1, HIDDEN), lambda i: (0, 0)),
            pl.BlockSpec((HIDDEN, HIDDEN), lambda i: (0, 0)),
        ],
        out_specs=pl.BlockSpec((MBLK, HIDDEN), lambda i: (i, 0)),
        out_shape=jax.ShapeDtypeStruct((N_NODES, HIDDEN), jnp.float32),
    )(agg1, y1, dinv, b1, W2)


def _head(agg2, y2, dinv, b2, W3, b3):
    return pl.pallas_call(
        _head_body,
        grid=(GRID,),
        in_specs=[
            pl.BlockSpec((NC, MBLK, HIDDEN), lambda i: (0, i, 0)),
            pl.BlockSpec((MBLK, HIDDEN), lambda i: (i, 0)),
            pl.BlockSpec((MBLK, 1), lambda i: (i, 0)),
            pl.BlockSpec((1, HIDDEN), lambda i: (0, 0)),
            pl.BlockSpec((HIDDEN, N_CLASSES), lambda i: (0, 0)),
            pl.BlockSpec((1, N_CLASSES), lambda i: (0, 0)),
        ],
        out_specs=pl.BlockSpec((MBLK, N_CLASSES), lambda i: (i, 0)),
        out_shape=jax.ShapeDtypeStruct((N_NODES, N_CLASSES), jnp.float32),
    )(agg2, y2, dinv, b2, W3, b3)


def kernel(x, edge_index, W1, b1, W2, b2, W3, b3):
    ei = edge_index.astype(jnp.int32)

    src1d = ei[0]
    dst1d = ei[1]
    degs = _deg_call()(dst1d)
    y1, dinv = _mm1(degs.T, x, W1)
    agg1 = _agg_call()(src1d, dst1d, y1)
    y2 = _mid(agg1, y1, dinv, b1.reshape(1, HIDDEN), W2)
    agg2 = _agg_call()(src1d, dst1d, y2)
    logits = _head(agg2, y2, dinv, b2.reshape(1, HIDDEN), W3,
                   b3.reshape(1, N_CLASSES))
    return logits
